# Initial kernel scaffold; baseline (speedup 1.0000x reference)
import jax
import jax.numpy as jnp
from jax.experimental import pallas as pl


def kernel(x, edge_index, edge_label_index, W1, b1, W2, b2, Wm1, bm1, Wm2, bm2):
    raise NotImplementedError("write your pallas kernel here")


# speedup vs baseline: 12.4241x; 12.4241x over previous
import functools

import jax
import jax.numpy as jnp
from jax import lax
from jax.experimental import pallas as pl
from jax.experimental.pallas import tpu as pltpu
from jax.experimental.pallas import tpu_sc as plsc

N = 10000
E = 320000
D_IN = 128
D_H = 128
D_OUT = 64

NHAT = 10240
EPAD = 327680
NW = 32
EPW = EPAD // NW
RPT = NHAT // 16

@functools.lru_cache(maxsize=None)
def _sc_mesh():
    return plsc.VectorSubcoreMesh(core_axis_name="c", subcore_axis_name="s")


def _zero_rows(buf, nrows, width):
    z16 = jnp.zeros((16,), jnp.float32)

    def body(i, _):
        for k in range(width // 16):
            buf[i, pl.ds(k * 16, 16)] = z16
        return _

    lax.fori_loop(0, nrows, body, None, unroll=False)


IDR = RPT // 128


def _fill_ids(ids, base):
    lane = lax.iota(jnp.int32, 16)

    def body(j, _):
        for k in range(8):
            ids[j, pl.ds(k * 16, 16)] = base + j * 128 + k * 16 + lane
        return _

    lax.fori_loop(0, IDR, body, None, unroll=False)


@functools.lru_cache(maxsize=None)
def _build_sc_degree():
    G = 2
    NCH = EPW // (G * 128)

    @functools.partial(
        pl.kernel,
        mesh=_sc_mesh(),
        out_type=jax.ShapeDtypeStruct((2 * NHAT, D_H), jnp.float32),
        scratch_types=[
            pltpu.VMEM((G, 128), jnp.int32),
            pltpu.VMEM((128, D_H), jnp.float32),
            pltpu.VMEM((128, D_H), jnp.float32),
            pltpu.VMEM((IDR, 128), jnp.int32),
            pltpu.VMEM_SHARED((NHAT, D_H), jnp.float32),
        ],
    )
    def deg_kernel(dst_hbm, out_hbm, dstv, ones_v, zbuf, ids, acc_sh):
        c = lax.axis_index("c")
        s = lax.axis_index("s")
        wid = c * 16 + s

        _fill_ids(ids, s * RPT)
        _zero_rows(zbuf, 128, D_H)
        one16 = jnp.ones((16,), jnp.float32)

        def ones_body(i, _):
            for k in range(D_H // 16):
                ones_v[i, pl.ds(k * 16, 16)] = one16
            return _

        lax.fori_loop(0, 128, ones_body, None, unroll=False)

        for j in range(IDR):
            pltpu.sync_copy(zbuf, acc_sh.at[ids.at[j]])
        plsc.subcore_barrier()

        def chunk(i, _):
            row0 = wid * (EPW // 128) + i * G
            pltpu.sync_copy(dst_hbm.at[pl.ds(row0, G)], dstv)
            for j in range(G):
                pltpu.sync_copy(ones_v, acc_sh.at[dstv.at[j]], add=True)
            return _

        lax.fori_loop(0, NCH, chunk, None, unroll=False)
        plsc.subcore_barrier()
        for j in range(IDR):
            pltpu.sync_copy(acc_sh.at[ids.at[j]], zbuf)
            pltpu.sync_copy(
                zbuf, out_hbm.at[pl.ds(c * NHAT + s * RPT + j * 128, 128)])

    return deg_kernel


def _sc_degree(dstp):
    return _build_sc_degree()(dstp).reshape(2, NHAT, D_H)


@functools.lru_cache(maxsize=None)
def _make_sc_mp(D):
    KC = 128
    PH = 2
    CPP = EPW // KC // PH

    @functools.partial(
        pl.kernel,
        mesh=_sc_mesh(),
        out_type=jax.ShapeDtypeStruct((2 * NHAT, D), jnp.float32),
        scratch_types=[
            pltpu.VMEM((CPP, 128), jnp.int32),
            pltpu.VMEM((CPP, 128), jnp.int32),
            pltpu.VMEM((KC, D), jnp.float32),
            pltpu.VMEM((KC, D), jnp.float32),
            pltpu.VMEM((IDR, 128), jnp.int32),
            pltpu.VMEM_SHARED((NHAT, D), jnp.float32),
            pltpu.SemaphoreType.DMA,
            pltpu.SemaphoreType.DMA,
        ],
    )
    def mp(y_hbm, src_hbm, dst_hbm, out_hbm, srcv, dstv, rowsA, rowsB, ids,
           acc_sh, semA, semB):
        c = lax.axis_index("c")
        s = lax.axis_index("s")
        wid = c * 16 + s

        _fill_ids(ids, s * RPT)
        _zero_rows(rowsA, 128, D)
        for j in range(IDR):
            pltpu.sync_copy(rowsA, acc_sh.at[ids.at[j]])
        plsc.subcore_barrier()

        def start(ci, buf, sem):
            return pltpu.async_copy(y_hbm.at[srcv.at[ci]], buf, sem)

        def wait(ci, buf, sem):
            pltpu.make_async_copy(y_hbm.at[srcv.at[ci]], buf, sem).wait()

        def scat(ci, buf):
            pltpu.sync_copy(buf, acc_sh.at[dstv.at[ci]], add=True)

        for ph in range(PH):
            base = wid * (EPW // 128) + ph * CPP
            pltpu.sync_copy(src_hbm.at[pl.ds(base, CPP)], srcv)
            pltpu.sync_copy(dst_hbm.at[pl.ds(base, CPP)], dstv)
            start(0, rowsA, semA)

            def pair(p, _):
                i0 = 2 * p
                start(i0 + 1, rowsB, semB)
                wait(i0, rowsA, semA)
                scat(i0, rowsA)

                @pl.when(p < CPP // 2 - 1)
                def _():
                    start(i0 + 2, rowsA, semA)

                wait(i0 + 1, rowsB, semB)
                scat(i0 + 1, rowsB)
                return _

            lax.fori_loop(0, CPP // 2, pair, None, unroll=False)

        plsc.subcore_barrier()
        for j in range(IDR):
            pltpu.sync_copy(acc_sh.at[ids.at[j]], rowsA)
            pltpu.sync_copy(
                rowsA, out_hbm.at[pl.ds(c * NHAT + s * RPT + j * 128, 128)])

    return mp


def _sc_mp128(y, srcp, dstp):
    return _make_sc_mp(D_H)(y, srcp, dstp).reshape(2, NHAT, D_H)


@functools.lru_cache(maxsize=None)
def _build_sc_decode_gather():
    KC = 128
    NCH = EPW // KC

    @functools.partial(
        pl.kernel,
        mesh=_sc_mesh(),
        out_type=jax.ShapeDtypeStruct((2 * EPAD, D_H), jnp.float32),
        scratch_types=[
            pltpu.VMEM((NCH, 128), jnp.int32),
            pltpu.VMEM((NCH, 128), jnp.int32),
            pltpu.VMEM((KC, D_H), jnp.float32),
            pltpu.VMEM((KC, D_H), jnp.float32),
            pltpu.VMEM((KC, D_H), jnp.float32),
            pltpu.VMEM((KC, D_H), jnp.float32),
            pltpu.SemaphoreType.DMA,
            pltpu.SemaphoreType.DMA,
        ],
    )
    def dec_kernel(z_hbm, e0_hbm, e1_hbm, out_hbm, e0v, e1v, r0A, r1A, r0B,
                   r1B, semA, semB):
        c = lax.axis_index("c")
        s = lax.axis_index("s")
        wid = c * 16 + s
        ibase = wid * (EPW // 128)
        pltpu.sync_copy(e0_hbm.at[pl.ds(ibase, NCH)], e0v)
        pltpu.sync_copy(e1_hbm.at[pl.ds(ibase, NCH)], e1v)

        def start(ci, b0, b1, sem):
            pltpu.async_copy(z_hbm.at[e0v.at[ci]], b0, sem)
            pltpu.async_copy(z_hbm.at[e1v.at[ci]], b1, sem)

        def finish(ci, b0, b1, sem):
            pltpu.make_async_copy(z_hbm.at[e0v.at[ci]], b0, sem).wait()
            pltpu.make_async_copy(z_hbm.at[e1v.at[ci]], b1, sem).wait()
            ebase = wid * EPW + ci * KC
            pltpu.sync_copy(b0, out_hbm.at[pl.ds(ebase, KC)])
            pltpu.sync_copy(b1, out_hbm.at[pl.ds(EPAD + ebase, KC)])

        start(0, r0A, r1A, semA)

        def pair(p, _):
            i0 = 2 * p
            start(i0 + 1, r0B, r1B, semB)
            finish(i0, r0A, r1A, semA)

            @pl.when(p < NCH // 2 - 1)
            def _():
                start(i0 + 2, r0A, r1A, semA)

            finish(i0 + 1, r0B, r1B, semB)
            return _

        lax.fori_loop(0, NCH // 2, pair, None, unroll=False)

    return dec_kernel


def _sc_decode_gather(z, e0p, e1p):
    return _build_sc_decode_gather()(z, e0p, e1p).reshape(2, EPAD, D_H)


def _dinv_body(degp_ref, o_ref):
    d = degp_ref[0] + degp_ref[1]
    o_ref[...] = lax.rsqrt(1.0 + d[:, 0:1])


def _y1_body(x_ref, w_ref, dinv_ref, o_ref):
    xw = jnp.dot(x_ref[...], w_ref[...], preferred_element_type=jnp.float32)
    o_ref[...] = xw * dinv_ref[...]


def _layer2_body(acc_ref, y1_ref, dinv_ref, b1_ref, w2_ref, o_ref):
    a = acc_ref[0] + acc_ref[1] + y1_ref[...]
    h = jnp.maximum(dinv_ref[...] * a + b1_ref[...], 0.0)
    hw = jnp.dot(h, w2_ref[...], preferred_element_type=jnp.float32)
    o_ref[...] = hw * dinv_ref[...]


def _z_body(acc_ref, y2_ref, dinv_ref, b2_ref, o_ref):
    a = acc_ref[0] + acc_ref[1] + y2_ref[...]
    o_ref[...] = dinv_ref[...] * a + b2_ref[...]


def _decode_body(ef_ref, wa_ref, wb_ref, bm1_ref, wm2_ref, bm2_ref, o_ref):
    hs = jnp.dot(ef_ref[0], wa_ref[...], preferred_element_type=jnp.float32)
    hd = jnp.dot(ef_ref[1], wb_ref[...], preferred_element_type=jnp.float32)
    h = jnp.maximum(hs + hd + bm1_ref[...], 0.0)
    o_ref[...] = jnp.dot(h, wm2_ref[...],
                         preferred_element_type=jnp.float32) + bm2_ref[...]


_NB = NHAT // 1024
_EB = 2048
_NEB = EPAD // _EB


def kernel(x, edge_index, edge_label_index, W1, b1, W2, b2, Wm1, bm1, Wm2,
           bm2):
    f32 = jnp.float32
    src = edge_index[0].astype(jnp.int32)
    dst = edge_index[1].astype(jnp.int32)
    e0 = edge_label_index[0].astype(jnp.int32)
    e1 = edge_label_index[1].astype(jnp.int32)

    npad = EPAD - E
    spread = jnp.arange(npad, dtype=jnp.int32)
    srcp = jnp.concatenate([src, spread % N]).reshape(EPAD // 128, 128)
    dstp = jnp.concatenate([dst, N + spread % (NHAT - N)])
    dstp = dstp.reshape(EPAD // 128, 128)
    e0p = jnp.concatenate([e0, spread % N]).reshape(EPAD // 128, 128)
    e1p = jnp.concatenate([e1, spread % N]).reshape(EPAD // 128, 128)

    xp = jnp.pad(x, ((0, NHAT - N), (0, 0)))
    b1r = b1.reshape(1, D_H)
    W2p = jnp.pad(W2, ((0, 0), (0, D_H - D_OUT)))
    b2r = jnp.pad(b2, (0, D_H - D_OUT)).reshape(1, D_H)
    Wm1a = jnp.pad(Wm1[:D_OUT], ((0, D_H - D_OUT), (0, 0)))
    Wm1b = jnp.pad(Wm1[D_OUT:], ((0, D_H - D_OUT), (0, 0)))
    bm1r = bm1.reshape(1, D_H)
    bm2r = bm2.reshape(1, 1)

    degp = _sc_degree(dstp)

    dinv = pl.pallas_call(
        _dinv_body,
        out_shape=jax.ShapeDtypeStruct((NHAT, 1), f32),
        in_specs=[pl.BlockSpec((2, NHAT, D_H), lambda: (0, 0, 0))],
        out_specs=pl.BlockSpec((NHAT, 1), lambda: (0, 0)),
    )(degp)

    y1 = pl.pallas_call(
        _y1_body,
        grid=(_NB,),
        out_shape=jax.ShapeDtypeStruct((NHAT, D_H), f32),
        in_specs=[
            pl.BlockSpec((1024, D_IN), lambda i: (i, 0)),
            pl.BlockSpec((D_IN, D_H), lambda i: (0, 0)),
            pl.BlockSpec((1024, 1), lambda i: (i, 0)),
        ],
        out_specs=pl.BlockSpec((1024, D_H), lambda i: (i, 0)),
    )(xp, W1, dinv)

    acc1 = _sc_mp128(y1, srcp, dstp)

    y2 = pl.pallas_call(
        _layer2_body,
        grid=(_NB,),
        out_shape=jax.ShapeDtypeStruct((NHAT, D_H), f32),
        in_specs=[
            pl.BlockSpec((2, 1024, D_H), lambda i: (0, i, 0)),
            pl.BlockSpec((1024, D_H), lambda i: (i, 0)),
            pl.BlockSpec((1024, 1), lambda i: (i, 0)),
            pl.BlockSpec((1, D_H), lambda i: (0, 0)),
            pl.BlockSpec((D_H, D_H), lambda i: (0, 0)),
        ],
        out_specs=pl.BlockSpec((1024, D_H), lambda i: (i, 0)),
    )(acc1, y1, dinv, b1r, W2p)

    acc2 = _sc_mp128(y2, srcp, dstp)

    z = pl.pallas_call(
        _z_body,
        grid=(_NB,),
        out_shape=jax.ShapeDtypeStruct((NHAT, D_H), f32),
        in_specs=[
            pl.BlockSpec((2, 1024, D_H), lambda i: (0, i, 0)),
            pl.BlockSpec((1024, D_H), lambda i: (i, 0)),
            pl.BlockSpec((1024, 1), lambda i: (i, 0)),
            pl.BlockSpec((1, D_H), lambda i: (0, 0)),
        ],
        out_specs=pl.BlockSpec((1024, D_H), lambda i: (i, 0)),
    )(acc2, y2, dinv, b2r)

    ef = _sc_decode_gather(z, e0p, e1p)

    dec = pl.pallas_call(
        _decode_body,
        grid=(_NEB,),
        out_shape=jax.ShapeDtypeStruct((EPAD, 1), f32),
        in_specs=[
            pl.BlockSpec((2, _EB, D_H), lambda i: (0, i, 0)),
            pl.BlockSpec((D_H, D_H), lambda i: (0, 0)),
            pl.BlockSpec((D_H, D_H), lambda i: (0, 0)),
            pl.BlockSpec((1, D_H), lambda i: (0, 0)),
            pl.BlockSpec((D_H, 1), lambda i: (0, 0)),
            pl.BlockSpec((1, 1), lambda i: (0, 0)),
        ],
        out_specs=pl.BlockSpec((_EB, 1), lambda i: (i, 0)),
    )(ef, Wm1a, Wm1b, bm1r, Wm2, bm2r)

    return dec[:E, 0]


# speedup vs baseline: 12.7355x; 1.0251x over previous
import functools

import jax
import jax.numpy as jnp
from jax import lax
from jax.experimental import pallas as pl
from jax.experimental.pallas import tpu as pltpu
from jax.experimental.pallas import tpu_sc as plsc

N = 10000
E = 320000
D_IN = 128
D_H = 128
D_OUT = 64

NHAT = 10240
EPAD = 327680
NW = 32
EPW = EPAD // NW
RPT = NHAT // 16

@functools.lru_cache(maxsize=None)
def _sc_mesh():
    return plsc.VectorSubcoreMesh(core_axis_name="c", subcore_axis_name="s")


def _zero_rows(buf, nrows, width):
    z16 = jnp.zeros((16,), jnp.float32)

    def body(i, _):
        for k in range(width // 16):
            buf[i, pl.ds(k * 16, 16)] = z16
        return _

    lax.fori_loop(0, nrows, body, None, unroll=False)


IDR = RPT // 128


def _fill_ids(ids, base):
    lane = lax.iota(jnp.int32, 16)

    def body(j, _):
        for k in range(8):
            ids[j, pl.ds(k * 16, 16)] = base + j * 128 + k * 16 + lane
        return _

    lax.fori_loop(0, IDR, body, None, unroll=False)


@functools.lru_cache(maxsize=None)
def _build_sc_degree():
    G = 4
    NCH = EPW // (G * 128)
    IB = EPW // 128

    @functools.partial(
        pl.kernel,
        mesh=_sc_mesh(),
        out_type=jax.ShapeDtypeStruct((2 * NHAT, D_H), jnp.float32),
        scratch_types=[
            pltpu.VMEM((IB, 128), jnp.int32),
            pltpu.VMEM((128, D_H), jnp.float32),
            pltpu.VMEM((IDR, 128), jnp.int32),
            pltpu.VMEM_SHARED((NHAT, D_H), jnp.float32),
            pltpu.SemaphoreType.DMA,
        ],
    )
    def deg_kernel(dst_hbm, out_hbm, dstv, buf, ids, acc_sh, sem):
        c = lax.axis_index("c")
        s = lax.axis_index("s")
        wid = c * 16 + s

        _fill_ids(ids, s * RPT)
        pltpu.sync_copy(dst_hbm.at[pl.ds(wid * IB, IB)], dstv)

        _zero_rows(buf, 128, D_H)
        for j in range(IDR):
            pltpu.sync_copy(buf, acc_sh.at[ids.at[j]])

        one16 = jnp.ones((16,), jnp.float32)

        def ones_body(i, _):
            for k in range(D_H // 16):
                buf[i, pl.ds(k * 16, 16)] = one16
            return _

        lax.fori_loop(0, 128, ones_body, None, unroll=False)
        plsc.subcore_barrier()

        def chunk(i, _):
            handles = [
                pltpu.async_copy(buf, acc_sh.at[dstv.at[i * G + j]], sem,
                                 add=True)
                for j in range(G)
            ]
            for h in handles:
                h.wait()
            return _

        lax.fori_loop(0, NCH, chunk, None, unroll=False)
        plsc.subcore_barrier()
        for j in range(IDR):
            pltpu.sync_copy(acc_sh.at[ids.at[j]], buf)
            pltpu.sync_copy(
                buf, out_hbm.at[pl.ds(c * NHAT + s * RPT + j * 128, 128)])

    return deg_kernel


def _sc_degree(dstp):
    return _build_sc_degree()(dstp).reshape(2, NHAT, D_H)


@functools.lru_cache(maxsize=None)
def _make_sc_mp(D):
    KC = 128
    PH = 2
    CPP = EPW // KC // PH

    @functools.partial(
        pl.kernel,
        mesh=_sc_mesh(),
        out_type=jax.ShapeDtypeStruct((2 * NHAT, D), jnp.float32),
        scratch_types=[
            pltpu.VMEM((CPP, 128), jnp.int32),
            pltpu.VMEM((CPP, 128), jnp.int32),
            pltpu.VMEM((KC, D), jnp.float32),
            pltpu.VMEM((KC, D), jnp.float32),
            pltpu.VMEM((IDR, 128), jnp.int32),
            pltpu.VMEM_SHARED((NHAT, D), jnp.float32),
            pltpu.SemaphoreType.DMA,
            pltpu.SemaphoreType.DMA,
        ],
    )
    def mp(y_hbm, src_hbm, dst_hbm, out_hbm, srcv, dstv, rowsA, rowsB, ids,
           acc_sh, semA, semB):
        c = lax.axis_index("c")
        s = lax.axis_index("s")
        wid = c * 16 + s

        _fill_ids(ids, s * RPT)
        _zero_rows(rowsA, 128, D)
        for j in range(IDR):
            pltpu.sync_copy(rowsA, acc_sh.at[ids.at[j]])
        plsc.subcore_barrier()

        def start(ci, buf, sem):
            return pltpu.async_copy(y_hbm.at[srcv.at[ci]], buf, sem)

        def wait(ci, buf, sem):
            pltpu.make_async_copy(y_hbm.at[srcv.at[ci]], buf, sem).wait()

        def scat(ci, buf):
            pltpu.sync_copy(buf, acc_sh.at[dstv.at[ci]], add=True)

        for ph in range(PH):
            base = wid * (EPW // 128) + ph * CPP
            pltpu.sync_copy(src_hbm.at[pl.ds(base, CPP)], srcv)
            pltpu.sync_copy(dst_hbm.at[pl.ds(base, CPP)], dstv)
            start(0, rowsA, semA)

            def pair(p, _):
                i0 = 2 * p
                start(i0 + 1, rowsB, semB)
                wait(i0, rowsA, semA)
                scat(i0, rowsA)

                @pl.when(p < CPP // 2 - 1)
                def _():
                    start(i0 + 2, rowsA, semA)

                wait(i0 + 1, rowsB, semB)
                scat(i0 + 1, rowsB)
                return _

            lax.fori_loop(0, CPP // 2, pair, None, unroll=False)

        plsc.subcore_barrier()
        for j in range(IDR):
            pltpu.sync_copy(acc_sh.at[ids.at[j]], rowsA)
            pltpu.sync_copy(
                rowsA, out_hbm.at[pl.ds(c * NHAT + s * RPT + j * 128, 128)])

    return mp


def _sc_mp128(y, srcp, dstp):
    return _make_sc_mp(D_H)(y, srcp, dstp).reshape(2, NHAT, D_H)


@functools.lru_cache(maxsize=None)
def _build_sc_decode_gather():
    KC = 128
    NCH = EPW // KC

    @functools.partial(
        pl.kernel,
        mesh=_sc_mesh(),
        out_type=jax.ShapeDtypeStruct((2 * EPAD, D_H), jnp.float32),
        scratch_types=[
            pltpu.VMEM((NCH, 128), jnp.int32),
            pltpu.VMEM((NCH, 128), jnp.int32),
            pltpu.VMEM((KC, D_H), jnp.float32),
            pltpu.VMEM((KC, D_H), jnp.float32),
            pltpu.VMEM((KC, D_H), jnp.float32),
            pltpu.VMEM((KC, D_H), jnp.float32),
            pltpu.SemaphoreType.DMA,
            pltpu.SemaphoreType.DMA,
        ],
    )
    def dec_kernel(z_hbm, e0_hbm, e1_hbm, out_hbm, e0v, e1v, r0A, r1A, r0B,
                   r1B, semA, semB):
        c = lax.axis_index("c")
        s = lax.axis_index("s")
        wid = c * 16 + s
        ibase = wid * (EPW // 128)
        pltpu.sync_copy(e0_hbm.at[pl.ds(ibase, NCH)], e0v)
        pltpu.sync_copy(e1_hbm.at[pl.ds(ibase, NCH)], e1v)

        def start(ci, b0, b1, sem):
            pltpu.async_copy(z_hbm.at[e0v.at[ci]], b0, sem)
            pltpu.async_copy(z_hbm.at[e1v.at[ci]], b1, sem)

        def finish(ci, b0, b1, sem):
            pltpu.make_async_copy(z_hbm.at[e0v.at[ci]], b0, sem).wait()
            pltpu.make_async_copy(z_hbm.at[e1v.at[ci]], b1, sem).wait()
            ebase = wid * EPW + ci * KC
            pltpu.sync_copy(b0, out_hbm.at[pl.ds(ebase, KC)])
            pltpu.sync_copy(b1, out_hbm.at[pl.ds(EPAD + ebase, KC)])

        start(0, r0A, r1A, semA)

        def pair(p, _):
            i0 = 2 * p
            start(i0 + 1, r0B, r1B, semB)
            finish(i0, r0A, r1A, semA)

            @pl.when(p < NCH // 2 - 1)
            def _():
                start(i0 + 2, r0A, r1A, semA)

            finish(i0 + 1, r0B, r1B, semB)
            return _

        lax.fori_loop(0, NCH // 2, pair, None, unroll=False)

    return dec_kernel


def _sc_decode_gather(z, e0p, e1p):
    return _build_sc_decode_gather()(z, e0p, e1p).reshape(2, EPAD, D_H)


def _y1_body(degp_ref, x_ref, w_ref, y_ref, dinv_ref):
    d = degp_ref[0, :, 0:1] + degp_ref[1, :, 0:1]
    dinv = lax.rsqrt(1.0 + d)
    dinv_ref[...] = dinv
    xw = jnp.dot(x_ref[...], w_ref[...], preferred_element_type=jnp.float32)
    y_ref[...] = xw * dinv


def _layer2_body(acc_ref, y1_ref, dinv_ref, b1_ref, w2_ref, o_ref):
    a = acc_ref[0] + acc_ref[1] + y1_ref[...]
    h = jnp.maximum(dinv_ref[...] * a + b1_ref[...], 0.0)
    hw = jnp.dot(h, w2_ref[...], preferred_element_type=jnp.float32)
    o_ref[...] = hw * dinv_ref[...]


def _z_body(acc_ref, y2_ref, dinv_ref, b2_ref, o_ref):
    a = acc_ref[0] + acc_ref[1] + y2_ref[...]
    o_ref[...] = dinv_ref[...] * a + b2_ref[...]


def _decode_body(ef_ref, wa_ref, wb_ref, bm1_ref, wm2_ref, bm2_ref, o_ref):
    hs = jnp.dot(ef_ref[0], wa_ref[...], preferred_element_type=jnp.float32)
    hd = jnp.dot(ef_ref[1], wb_ref[...], preferred_element_type=jnp.float32)
    h = jnp.maximum(hs + hd + bm1_ref[...], 0.0)
    o_ref[...] = jnp.dot(h, wm2_ref[...],
                         preferred_element_type=jnp.float32) + bm2_ref[...]


_NB = NHAT // 1024
_EB = 2048
_NEB = EPAD // _EB


def kernel(x, edge_index, edge_label_index, W1, b1, W2, b2, Wm1, bm1, Wm2,
           bm2):
    f32 = jnp.float32
    src = edge_index[0].astype(jnp.int32)
    dst = edge_index[1].astype(jnp.int32)
    e0 = edge_label_index[0].astype(jnp.int32)
    e1 = edge_label_index[1].astype(jnp.int32)

    npad = EPAD - E
    spread = jnp.arange(npad, dtype=jnp.int32)
    srcp = jnp.concatenate([src, spread % N]).reshape(EPAD // 128, 128)
    dstp = jnp.concatenate([dst, N + spread % (NHAT - N)])
    dstp = dstp.reshape(EPAD // 128, 128)
    e0p = jnp.concatenate([e0, spread % N]).reshape(EPAD // 128, 128)
    e1p = jnp.concatenate([e1, spread % N]).reshape(EPAD // 128, 128)

    xp = jnp.pad(x, ((0, NHAT - N), (0, 0)))
    b1r = b1.reshape(1, D_H)
    W2p = jnp.pad(W2, ((0, 0), (0, D_H - D_OUT)))
    b2r = jnp.pad(b2, (0, D_H - D_OUT)).reshape(1, D_H)
    Wm1a = jnp.pad(Wm1[:D_OUT], ((0, D_H - D_OUT), (0, 0)))
    Wm1b = jnp.pad(Wm1[D_OUT:], ((0, D_H - D_OUT), (0, 0)))
    bm1r = bm1.reshape(1, D_H)
    bm2r = bm2.reshape(1, 1)

    degp = _sc_degree(dstp)

    y1, dinv = pl.pallas_call(
        _y1_body,
        grid=(_NB,),
        out_shape=(jax.ShapeDtypeStruct((NHAT, D_H), f32),
                   jax.ShapeDtypeStruct((NHAT, 1), f32)),
        in_specs=[
            pl.BlockSpec((2, 1024, D_H), lambda i: (0, i, 0)),
            pl.BlockSpec((1024, D_IN), lambda i: (i, 0)),
            pl.BlockSpec((D_IN, D_H), lambda i: (0, 0)),
        ],
        out_specs=[pl.BlockSpec((1024, D_H), lambda i: (i, 0)),
                   pl.BlockSpec((1024, 1), lambda i: (i, 0))],
    )(degp, xp, W1)

    acc1 = _sc_mp128(y1, srcp, dstp)

    y2 = pl.pallas_call(
        _layer2_body,
        grid=(_NB,),
        out_shape=jax.ShapeDtypeStruct((NHAT, D_H), f32),
        in_specs=[
            pl.BlockSpec((2, 1024, D_H), lambda i: (0, i, 0)),
            pl.BlockSpec((1024, D_H), lambda i: (i, 0)),
            pl.BlockSpec((1024, 1), lambda i: (i, 0)),
            pl.BlockSpec((1, D_H), lambda i: (0, 0)),
            pl.BlockSpec((D_H, D_H), lambda i: (0, 0)),
        ],
        out_specs=pl.BlockSpec((1024, D_H), lambda i: (i, 0)),
    )(acc1, y1, dinv, b1r, W2p)

    acc2 = _sc_mp128(y2, srcp, dstp)

    z = pl.pallas_call(
        _z_body,
        grid=(_NB,),
        out_shape=jax.ShapeDtypeStruct((NHAT, D_H), f32),
        in_specs=[
            pl.BlockSpec((2, 1024, D_H), lambda i: (0, i, 0)),
            pl.BlockSpec((1024, D_H), lambda i: (i, 0)),
            pl.BlockSpec((1024, 1), lambda i: (i, 0)),
            pl.BlockSpec((1, D_H), lambda i: (0, 0)),
        ],
        out_specs=pl.BlockSpec((1024, D_H), lambda i: (i, 0)),
    )(acc2, y2, dinv, b2r)

    ef = _sc_decode_gather(z, e0p, e1p)

    dec = pl.pallas_call(
        _decode_body,
        grid=(_NEB,),
        out_shape=jax.ShapeDtypeStruct((EPAD, 1), f32),
        in_specs=[
            pl.BlockSpec((2, _EB, D_H), lambda i: (0, i, 0)),
            pl.BlockSpec((D_H, D_H), lambda i: (0, 0)),
            pl.BlockSpec((D_H, D_H), lambda i: (0, 0)),
            pl.BlockSpec((1, D_H), lambda i: (0, 0)),
            pl.BlockSpec((D_H, 1), lambda i: (0, 0)),
            pl.BlockSpec((1, 1), lambda i: (0, 0)),
        ],
        out_specs=pl.BlockSpec((_EB, 1), lambda i: (i, 0)),
    )(ef, Wm1a, Wm1b, bm1r, Wm2, bm2r)

    return dec[:E, 0]
